# baseline (speedup 1.0000x reference)
import jax
import jax.numpy as jnp
from jax import lax
from jax.experimental import pallas as pl
from jax.experimental.pallas import tpu as pltpu
from jax.experimental.pallas import tpu_sc as plsc

VOCAB = 1000
B = 4096
T = 20
NC = 2
NS = 16
NW = NC * NS
BATCHES_PER_W = B // NW
NB = 4

STAGE_ROWS = VOCAB // NS
STAGE_REM = VOCAB - STAGE_ROWS * NS


def _gather_body(table_hbm, idx_hbm, out_hbm, idx_v, rows0, rows1, rows2,
                 rows3, gs0, gs1, gs2, gs3, ws0, ws1, ws2, ws3):
    rows = (rows0, rows1, rows2, rows3)
    gsem = (gs0, gs1, gs2, gs3)
    wsem = (ws0, ws1, ws2, ws3)

    sid = lax.axis_index("s")
    wid = sid * NC + lax.axis_index("c")
    base = wid * BATCHES_PER_W
    pltpu.sync_copy(idx_hbm.at[wid], idx_v)

    def gather(j, b):
        return pltpu.make_async_copy(
            table_hbm.at[idx_v.at[j]], rows[b], gsem[b]
        )

    def write(j, b):
        return pltpu.make_async_copy(rows[b], out_hbm.at[base + j], wsem[b])

    for b in range(NB - 1):
        gather(b, b).start()

    def group(g, carry):
        for b in range(NB):
            j = g * NB + b
            bn = (b + NB - 1) % NB
            if b == 0:
                @pl.when(g >= 1)
                def _():
                    write(j - 1, bn).wait()
                    gather(j + NB - 1, bn).start()

                @pl.when(g == 0)
                def _():
                    gather(j + NB - 1, bn).start()
            else:
                write(j - 1, bn).wait()

                @pl.when(j + NB - 1 < BATCHES_PER_W)
                def _():
                    gather(j + NB - 1, bn).start()
            gather(j, b).wait()
            write(j, b).start()
        return carry

    lax.fori_loop(0, BATCHES_PER_W // NB, group, 0)
    write(BATCHES_PER_W - 1, (BATCHES_PER_W - 1) % NB).wait()


@jax.jit
def _run(idx3, table):
    mesh = plsc.VectorSubcoreMesh(core_axis_name="c", subcore_axis_name="s")
    return pl.kernel(
        _gather_body,
        out_type=jax.ShapeDtypeStruct((B, T, VOCAB), jnp.float32),
        mesh=mesh,
        scratch_types=[
            pltpu.VMEM((BATCHES_PER_W, T), jnp.int32),
            pltpu.VMEM((T, VOCAB), jnp.float32),
            pltpu.VMEM((T, VOCAB), jnp.float32),
            pltpu.VMEM((T, VOCAB), jnp.float32),
            pltpu.VMEM((T, VOCAB), jnp.float32),
            pltpu.SemaphoreType.DMA,
            pltpu.SemaphoreType.DMA,
            pltpu.SemaphoreType.DMA,
            pltpu.SemaphoreType.DMA,
            pltpu.SemaphoreType.DMA,
            pltpu.SemaphoreType.DMA,
            pltpu.SemaphoreType.DMA,
            pltpu.SemaphoreType.DMA,
        ],
        compiler_params=pltpu.CompilerParams(use_tc_tiling_on_sc=False),
    )(table, idx3)


def kernel(idx, token_embedding_table):
    idx3 = idx.reshape(NW, BATCHES_PER_W, T)
    return _run(idx3, token_embedding_table)
